# Initial kernel scaffold; baseline (speedup 1.0000x reference)
import jax
import jax.numpy as jnp
from jax.experimental import pallas as pl


def kernel(video_features, audio_features, video_snippet_preds, audio_snippet_preds, confidence_threshold, aW0, a_src0, a_trg0, a_b0, vW0, v_src0, v_trg0, v_b0):
    raise NotImplementedError("write your pallas kernel here")


# speedup vs baseline: 2.2493x; 2.2493x over previous
import jax
import jax.numpy as jnp
from jax.experimental import pallas as pl
from jax.experimental.pallas import tpu as pltpu

B, S, F = 64, 25, 256
NE, NH = 10, 4


def _gat_body(x_ref, preds_ref, thr_ref, w_ref, asrc_ref, atrg_ref, b_ref, o_ref):
    x = x_ref[0, 0]
    w = w_ref[0]
    proj = jnp.dot(x, w, preferred_element_type=jnp.float32)

    thr = thr_ref[0, 0]
    m = preds_ref[0, 0] >= thr

    row = jax.lax.broadcasted_iota(jnp.int32, (S, S), 0)
    col = jax.lax.broadcasted_iota(jnp.int32, (S, S), 1)
    eye = row == col
    base = (jnp.abs(row - col) == 1) | eye
    noteye = jnp.logical_not(eye)

    adjs = []
    for i in range(NE):
        mi = m[:, i]
        adjs.append(base | (mi[:, None] & mi[None, :] & noteye))

    prod_s = proj * asrc_ref[0]
    prod_t = proj * atrg_ref[0]

    acc = jnp.zeros((S, F), jnp.float32)
    for h in range(NH):
        ph = proj[:, h * F:(h + 1) * F]
        ss = jnp.sum(prod_s[:, h * F:(h + 1) * F], axis=1)
        st = jnp.sum(prod_t[:, h * F:(h + 1) * F], axis=1)
        sc = ss[:, None] + st[None, :]
        sc = jnp.where(sc >= 0, sc, 0.2 * sc)
        cmax = jnp.max(sc, axis=0)
        e = jnp.exp(sc - cmax[None, :])
        s_acc = jnp.zeros((S, S), jnp.float32)
        for i in range(NE):
            adj = adjs[i]
            den = jnp.sum(jnp.where(adj, e, 0.0), axis=0)
            s_acc = s_acc + jnp.where(adj, (1.0 / den)[None, :], 0.0)
        atot = e * s_acc
        acc = acc + jax.lax.dot_general(
            atot, ph, (((0,), (0,)), ((), ())),
            preferred_element_type=jnp.float32)

    o_ref[0, 0] = x + b_ref[0] + acc * (1.0 / (NE * NH))


def kernel(video_features, audio_features, video_snippet_preds,
           audio_snippet_preds, confidence_threshold, aW0, a_src0, a_trg0,
           a_b0, vW0, v_src0, v_trg0, v_b0):
    xs = jnp.stack([video_features, audio_features])
    preds = jnp.stack([video_snippet_preds, audio_snippet_preds])
    thr = jnp.asarray(confidence_threshold, jnp.float32).reshape(1, 1)
    ws = jnp.stack([vW0, aW0])
    asrc = jnp.stack([v_src0.reshape(1, NH * F), a_src0.reshape(1, NH * F)])
    atrg = jnp.stack([v_trg0.reshape(1, NH * F), a_trg0.reshape(1, NH * F)])
    bs = jnp.stack([v_b0.reshape(1, F), a_b0.reshape(1, F)])

    out = pl.pallas_call(
        _gat_body,
        grid=(2, B),
        in_specs=[
            pl.BlockSpec((1, 1, S, F), lambda mo, b: (mo, b, 0, 0)),
            pl.BlockSpec((1, 1, S, NE), lambda mo, b: (mo, b, 0, 0)),
            pl.BlockSpec((1, 1), lambda mo, b: (0, 0)),
            pl.BlockSpec((1, F, NH * F), lambda mo, b: (mo, 0, 0)),
            pl.BlockSpec((1, 1, NH * F), lambda mo, b: (mo, 0, 0)),
            pl.BlockSpec((1, 1, NH * F), lambda mo, b: (mo, 0, 0)),
            pl.BlockSpec((1, 1, F), lambda mo, b: (mo, 0, 0)),
        ],
        out_specs=pl.BlockSpec((1, 1, S, F), lambda mo, b: (mo, b, 0, 0)),
        out_shape=jax.ShapeDtypeStruct((2, B, S, F), jnp.float32),
    )(xs, preds, thr, ws, asrc, atrg, bs)
    return (out[0], out[1])
